# baseline (speedup 1.0000x reference)
import functools

import jax
import jax.numpy as jnp
from jax import lax
from jax.experimental import pallas as pl
from jax.experimental.pallas import tpu as pltpu
from jax.experimental.pallas import tpu_sc as plsc

_MAXG = 5976
_T, _B, _R, _C, _D, _NINV = 16, 32, 21, 79, 64, 55
_NTB = _T * _B
_CP = 80
_GW = _R * _CP
_META0 = _GW
_INVO = _GW + 16
_PACKW = 1792
_NC, _NS = 2, 16
_DPT = 8
_NDB = _D // _DPT
_ETW = 5984
_RP, _CT = 24, 128
_SROWS = _DPT * _RP
_VIC0, _SELF0, _INV0 = 0, 9, 16
_NTILE = _NTB // (_NC * 2)


def _compute_tb(lane, in_v, et_v, sbuf, small_v):
    @plsc.parallel_loop(0, _R, unroll=2)
    def row(r):
        for c in range(_CP // 16):
            g = in_v[pl.ds(r * _CP + c * 16, 16)]
            for dl in range(_DPT):
                v = plsc.load_gather(et_v, [g + dl * _ETW])
                sbuf[dl * _RP + r, pl.ds(c * 16, 16)] = v

    xv = plsc.load_gather(in_v, [jnp.full((16,), _META0, jnp.int32)])
    yv = plsc.load_gather(in_v, [jnp.full((16,), _META0 + 1, jnp.int32)])
    ii = lane // 3
    jj = lane - ii * 3
    rr = yv - 1 + ii
    cc = xv - 1 + jj
    inb = (rr >= 0) & (rr < _R) & (cc >= 0) & (cc < _C) & (lane < 9)
    flat = jnp.where(inb, rr * _CP + cc, 0)
    gl = plsc.load_gather(in_v, [flat])
    vic = jnp.where(inb, gl, _MAXG)

    for dl in range(_DPT):
        vv = plsc.load_gather(et_v, [vic + dl * _ETW])
        small_v[dl, pl.ds(0, 16)] = vv
        plsc.store_scatter(
            small_v,
            [jnp.full((16,), dl, jnp.int32), jnp.full((16,), _SELF0, jnp.int32)],
            vv,
            mask=lane == 4,
        )

    ivs = [in_v[pl.ds(_INVO + c2 * 16, 16)] for c2 in range(4)]
    for dl in range(_DPT):
        for c2 in range(4):
            vv = plsc.load_gather(et_v, [ivs[c2] + dl * _ETW])
            small_v[dl, pl.ds(_INV0 + c2 * 16, 16)] = vv


def _sc_body(
    pack_hbm, et_hbm, stage_hbm, small_hbm,
    in_a, in_b, et_v, sb_a, sb_b, sm_a, sm_b,
    si_a, si_b, ss_a, ss_b, sq_a, sq_b,
):
    cid = lax.axis_index("c")
    tid = lax.axis_index("s")
    dblk = tid % _NDB
    grp = tid // _NDB
    tb0 = (cid * 2 + grp) * _NTILE
    lane = lax.iota(jnp.int32, 16)

    pltpu.sync_copy(et_hbm.at[pl.ds(dblk * _DPT * _ETW, _DPT * _ETW)], et_v)

    bufs = ((in_a, sb_a, sm_a, si_a, ss_a, sq_a), (in_b, sb_b, sm_b, si_b, ss_b, sq_b))

    def in_src(tb):
        return pack_hbm.at[pl.ds(tb * _PACKW, _PACKW)]

    def stage_dst(tb):
        return stage_hbm.at[tb, pl.ds(dblk * _SROWS, _SROWS)]

    pltpu.async_copy(in_src(tb0), in_a, si_a)
    pltpu.async_copy(in_src(tb0 + 1), in_b, si_b)

    @pl.loop(0, _NTILE, step=2)
    def pair(i):
        for b in range(2):
            in_v, sbuf, small_v, si, ss, sq = bufs[b]
            g = i + b
            tb = tb0 + g
            pltpu.make_async_copy(in_src(tb), in_v, si).wait()

            @pl.when(g >= 2)
            def _wait_prev_out():
                pltpu.make_async_copy(sbuf, stage_dst(tb), ss).wait()
                pltpu.make_async_copy(small_v, small_hbm.at[tb, dblk], sq).wait()

            _compute_tb(lane, in_v, et_v, sbuf, small_v)

            pltpu.async_copy(sbuf, stage_dst(tb), ss)
            pltpu.async_copy(small_v, small_hbm.at[tb, dblk], sq)

            @pl.when(g + 2 < _NTILE)
            def _prefetch_next():
                pltpu.async_copy(in_src(tb + 2), in_v, si)

    for b in range(2):
        in_v, sbuf, small_v, si, ss, sq = bufs[b]
        tb = tb0 + _NTILE - 2 + b
        pltpu.make_async_copy(sbuf, stage_dst(tb), ss).wait()
        pltpu.make_async_copy(small_v, small_hbm.at[tb, dblk], sq).wait()


def _tc_depad_body(in_ref, out_ref):
    out_ref[0, 0] = in_ref[:, :_R, :_C]


def _tc_depad(stage):
    planes = stage.reshape(_NTB * _D, _RP, _CT)
    return pl.pallas_call(
        _tc_depad_body,
        grid=(_NTB,),
        in_specs=[
            pl.BlockSpec((_D, _RP, _CT), lambda tb: (tb, 0, 0)),
        ],
        out_specs=pl.BlockSpec(
            (1, 1, _D, _R, _C),
            lambda tb: (tb // _B, tb % _B, 0, 0, 0),
        ),
        out_shape=jax.ShapeDtypeStruct((_T, _B, _D, _R, _C), jnp.float32),
    )(planes)


def kernel(glyphs, blstats, inv_glyphs, emb_table):
    gly = glyphs.reshape(_NTB, _R, _C).astype(jnp.int32)
    gly = jnp.pad(gly, ((0, 0), (0, 0), (0, _CP - _C)), constant_values=_MAXG)
    bl = blstats.reshape(_NTB, blstats.shape[-1]).astype(jnp.int32)
    inv = inv_glyphs.reshape(_NTB, _NINV).astype(jnp.int32)

    pack = jnp.full((_NTB, _PACKW), _MAXG, jnp.int32)
    pack = pack.at[:, :_GW].set(gly.reshape(_NTB, _GW))
    pack = pack.at[:, _META0].set(bl[:, 0])
    pack = pack.at[:, _META0 + 1].set(bl[:, 1])
    pack = pack.at[:, _INVO : _INVO + _NINV].set(inv)
    pack = pack.reshape(_NTB * _PACKW)

    et = jnp.zeros((_D, _ETW), jnp.float32)
    et = et.at[:, : _MAXG + 1].set(emb_table.astype(jnp.float32).T)
    et = et.reshape(_D * _ETW)

    mesh = plsc.VectorSubcoreMesh(
        core_axis_name="c", subcore_axis_name="s", num_cores=_NC, num_subcores=_NS
    )
    run = functools.partial(
        pl.kernel,
        out_type=[
            jax.ShapeDtypeStruct((_NTB, _NDB * _SROWS, _CT), jnp.float32),
            jax.ShapeDtypeStruct((_NTB, _NDB, _DPT, _CT), jnp.float32),
        ],
        mesh=mesh,
        compiler_params=pltpu.CompilerParams(
            needs_layout_passes=False, use_tc_tiling_on_sc=True
        ),
        scratch_types=[
            pltpu.VMEM((_PACKW,), jnp.int32),
            pltpu.VMEM((_PACKW,), jnp.int32),
            pltpu.VMEM((_DPT * _ETW,), jnp.float32),
            pltpu.VMEM((_SROWS, _CT), jnp.float32),
            pltpu.VMEM((_SROWS, _CT), jnp.float32),
            pltpu.VMEM((_DPT, _CT), jnp.float32),
            pltpu.VMEM((_DPT, _CT), jnp.float32),
            pltpu.SemaphoreType.DMA,
            pltpu.SemaphoreType.DMA,
            pltpu.SemaphoreType.DMA,
            pltpu.SemaphoreType.DMA,
            pltpu.SemaphoreType.DMA,
            pltpu.SemaphoreType.DMA,
        ],
    )(_sc_body)
    stage, small_f = run(pack, et)

    screen = _tc_depad(stage)
    vicinity = small_f[:, :, :, :_SELF0].reshape(_T, _B, _D, 3, 3)
    self_ = small_f[:, :, :, _SELF0].reshape(_T, _B, _D)
    inventory = small_f[:, :, :, _INV0 : _INV0 + _NINV].reshape(_T, _B, _D, _NINV)
    return screen, vicinity, inventory, self_


# speedup vs baseline: 1.2229x; 1.2229x over previous
import functools

import jax
import jax.numpy as jnp
from jax import lax
from jax.experimental import pallas as pl
from jax.experimental.pallas import tpu as pltpu
from jax.experimental.pallas import tpu_sc as plsc

_MAXG = 5976
_T, _B, _R, _C, _D, _NINV = 16, 32, 21, 79, 64, 55
_NTB = _T * _B
_CP = 80
_GW = _R * _CP
_META0 = _GW
_INVO = _GW + 16
_PACKW = 1792
_NC, _NS = 2, 16
_DPT = 8
_NDB = _D // _DPT
_ETW = 5984
_RP, _CT = 24, 128
_SROWS = _DPT * _RP
_VIC0, _SELF0, _INV0 = 0, 9, 16
_NTILE = _NTB // (_NC * 2)


def _compute_tb(lane, in_v, et_v, sbuf, small_v):
    @plsc.parallel_loop(0, _R, unroll=2)
    def row(r):
        for c in range(_CP // 16):
            g = in_v[pl.ds(r * _CP + c * 16, 16)]
            for dl in range(_DPT):
                v = plsc.load_gather(et_v, [g + dl * _ETW])
                sbuf[dl * _RP + r, pl.ds(c * 16, 16)] = v

    xv = plsc.load_gather(in_v, [jnp.full((16,), _META0, jnp.int32)])
    yv = plsc.load_gather(in_v, [jnp.full((16,), _META0 + 1, jnp.int32)])
    ii = lane // 3
    jj = lane - ii * 3
    rr = yv - 1 + ii
    cc = xv - 1 + jj
    inb = (rr >= 0) & (rr < _R) & (cc >= 0) & (cc < _C) & (lane < 9)
    flat = jnp.where(inb, rr * _CP + cc, 0)
    gl = plsc.load_gather(in_v, [flat])
    vic = jnp.where(inb, gl, _MAXG)

    for dl in range(_DPT):
        vv = plsc.load_gather(et_v, [vic + dl * _ETW])
        small_v[dl, pl.ds(0, 16)] = vv
        plsc.store_scatter(
            small_v,
            [jnp.full((16,), dl, jnp.int32), jnp.full((16,), _SELF0, jnp.int32)],
            vv,
            mask=lane == 4,
        )

    ivs = [in_v[pl.ds(_INVO + c2 * 16, 16)] for c2 in range(4)]
    for dl in range(_DPT):
        for c2 in range(4):
            vv = plsc.load_gather(et_v, [ivs[c2] + dl * _ETW])
            small_v[dl, pl.ds(_INV0 + c2 * 16, 16)] = vv


def _sc_body(
    pack_hbm, et_hbm, stage_hbm, small_hbm,
    in_a, in_b, et_v, sb_a, sb_b, sm_a, sm_b,
    si_a, si_b, ss_a, ss_b, sq_a, sq_b,
):
    cid = lax.axis_index("c")
    tid = lax.axis_index("s")
    dblk = tid % _NDB
    grp = tid // _NDB
    tb0 = (cid * 2 + grp) * _NTILE
    lane = lax.iota(jnp.int32, 16)

    pltpu.sync_copy(et_hbm.at[pl.ds(dblk * _DPT * _ETW, _DPT * _ETW)], et_v)

    bufs = ((in_a, sb_a, sm_a, si_a, ss_a, sq_a), (in_b, sb_b, sm_b, si_b, ss_b, sq_b))

    def in_src(tb):
        return pack_hbm.at[pl.ds(tb * _PACKW, _PACKW)]

    def stage_dst(tb):
        return stage_hbm.at[tb, pl.ds(dblk * _SROWS, _SROWS)]

    pltpu.async_copy(in_src(tb0), in_a, si_a)
    pltpu.async_copy(in_src(tb0 + 1), in_b, si_b)

    @pl.loop(0, _NTILE, step=2)
    def pair(i):
        for b in range(2):
            in_v, sbuf, small_v, si, ss, sq = bufs[b]
            g = i + b
            tb = tb0 + g
            pltpu.make_async_copy(in_src(tb), in_v, si).wait()

            @pl.when(g >= 2)
            def _wait_prev_out():
                pltpu.make_async_copy(sbuf, stage_dst(tb), ss).wait()
                pltpu.make_async_copy(small_v, small_hbm.at[tb, dblk], sq).wait()

            _compute_tb(lane, in_v, et_v, sbuf, small_v)

            pltpu.async_copy(sbuf, stage_dst(tb), ss)
            pltpu.async_copy(small_v, small_hbm.at[tb, dblk], sq)

            @pl.when(g + 2 < _NTILE)
            def _prefetch_next():
                pltpu.async_copy(in_src(tb + 2), in_v, si)

    for b in range(2):
        in_v, sbuf, small_v, si, ss, sq = bufs[b]
        tb = tb0 + _NTILE - 2 + b
        pltpu.make_async_copy(sbuf, stage_dst(tb), ss).wait()
        pltpu.make_async_copy(small_v, small_hbm.at[tb, dblk], sq).wait()


_TBS = 4


def _tc_depad_body(in_ref, out_ref):
    x = in_ref[:, :_R, :_C]
    out_ref[0] = x.reshape(_TBS, _D, _R, _C)


def _tc_depad(stage):
    planes = stage.reshape(_NTB * _D, _RP, _CT)
    return pl.pallas_call(
        _tc_depad_body,
        grid=(_NTB // _TBS,),
        in_specs=[
            pl.BlockSpec((_TBS * _D, _RP, _CT), lambda i: (i, 0, 0)),
        ],
        out_specs=pl.BlockSpec(
            (1, _TBS, _D, _R, _C),
            lambda i: ((i * _TBS) // _B, (i * _TBS) % _B // _TBS, 0, 0, 0),
        ),
        out_shape=jax.ShapeDtypeStruct((_T, _B, _D, _R, _C), jnp.float32),
    )(planes)


def kernel(glyphs, blstats, inv_glyphs, emb_table):
    gly = glyphs.reshape(_NTB, _R, _C).astype(jnp.int32)
    gly = jnp.pad(gly, ((0, 0), (0, 0), (0, _CP - _C)), constant_values=_MAXG)
    bl = blstats.reshape(_NTB, blstats.shape[-1]).astype(jnp.int32)
    inv = inv_glyphs.reshape(_NTB, _NINV).astype(jnp.int32)

    pack = jnp.full((_NTB, _PACKW), _MAXG, jnp.int32)
    pack = pack.at[:, :_GW].set(gly.reshape(_NTB, _GW))
    pack = pack.at[:, _META0].set(bl[:, 0])
    pack = pack.at[:, _META0 + 1].set(bl[:, 1])
    pack = pack.at[:, _INVO : _INVO + _NINV].set(inv)
    pack = pack.reshape(_NTB * _PACKW)

    et = jnp.zeros((_D, _ETW), jnp.float32)
    et = et.at[:, : _MAXG + 1].set(emb_table.astype(jnp.float32).T)
    et = et.reshape(_D * _ETW)

    mesh = plsc.VectorSubcoreMesh(
        core_axis_name="c", subcore_axis_name="s", num_cores=_NC, num_subcores=_NS
    )
    run = functools.partial(
        pl.kernel,
        out_type=[
            jax.ShapeDtypeStruct((_NTB, _NDB * _SROWS, _CT), jnp.float32),
            jax.ShapeDtypeStruct((_NTB, _NDB, _DPT, _CT), jnp.float32),
        ],
        mesh=mesh,
        compiler_params=pltpu.CompilerParams(
            needs_layout_passes=False, use_tc_tiling_on_sc=True
        ),
        scratch_types=[
            pltpu.VMEM((_PACKW,), jnp.int32),
            pltpu.VMEM((_PACKW,), jnp.int32),
            pltpu.VMEM((_DPT * _ETW,), jnp.float32),
            pltpu.VMEM((_SROWS, _CT), jnp.float32),
            pltpu.VMEM((_SROWS, _CT), jnp.float32),
            pltpu.VMEM((_DPT, _CT), jnp.float32),
            pltpu.VMEM((_DPT, _CT), jnp.float32),
            pltpu.SemaphoreType.DMA,
            pltpu.SemaphoreType.DMA,
            pltpu.SemaphoreType.DMA,
            pltpu.SemaphoreType.DMA,
            pltpu.SemaphoreType.DMA,
            pltpu.SemaphoreType.DMA,
        ],
    )(_sc_body)
    stage, small_f = run(pack, et)

    screen = _tc_depad(stage)
    vicinity = small_f[:, :, :, :_SELF0].reshape(_T, _B, _D, 3, 3)
    self_ = small_f[:, :, :, _SELF0].reshape(_T, _B, _D)
    inventory = small_f[:, :, :, _INV0 : _INV0 + _NINV].reshape(_T, _B, _D, _NINV)
    return screen, vicinity, inventory, self_


# speedup vs baseline: 1.2258x; 1.0024x over previous
import functools

import jax
import jax.numpy as jnp
from jax import lax
from jax.experimental import pallas as pl
from jax.experimental.pallas import tpu as pltpu
from jax.experimental.pallas import tpu_sc as plsc

_MAXG = 5976
_T, _B, _R, _C, _D, _NINV = 16, 32, 21, 79, 64, 55
_NTB = _T * _B
_CP = 80
_GW = _R * _CP
_META0 = _GW
_INVO = _GW + 16
_PACKW = 1792
_NC, _NS = 2, 16
_DPT = 8
_NDB = _D // _DPT
_ETW = 5984
_RP, _CT = 24, 128
_SROWS = _DPT * _RP
_VIC0, _SELF0, _INV0 = 0, 9, 16
_NTILE = _NTB // (_NC * 2)


def _compute_tb(lane, in_v, et_v, sbuf, small_v):
    @plsc.parallel_loop(0, _R, unroll=2)
    def row(r):
        for c in range(_CP // 16):
            g = in_v[pl.ds(r * _CP + c * 16, 16)]
            for dl in range(_DPT):
                v = plsc.load_gather(et_v, [g + dl * _ETW])
                sbuf[dl * _RP + r, pl.ds(c * 16, 16)] = v

    xv = plsc.load_gather(in_v, [jnp.full((16,), _META0, jnp.int32)])
    yv = plsc.load_gather(in_v, [jnp.full((16,), _META0 + 1, jnp.int32)])
    ii = lane // 3
    jj = lane - ii * 3
    rr = yv - 1 + ii
    cc = xv - 1 + jj
    inb = (rr >= 0) & (rr < _R) & (cc >= 0) & (cc < _C) & (lane < 9)
    flat = jnp.where(inb, rr * _CP + cc, 0)
    gl = plsc.load_gather(in_v, [flat])
    vic = jnp.where(inb, gl, _MAXG)

    for dl in range(_DPT):
        vv = plsc.load_gather(et_v, [vic + dl * _ETW])
        small_v[dl, pl.ds(0, 16)] = vv
        plsc.store_scatter(
            small_v,
            [jnp.full((16,), dl, jnp.int32), jnp.full((16,), _SELF0, jnp.int32)],
            vv,
            mask=lane == 4,
        )

    ivs = [in_v[pl.ds(_INVO + c2 * 16, 16)] for c2 in range(4)]
    for dl in range(_DPT):
        for c2 in range(4):
            vv = plsc.load_gather(et_v, [ivs[c2] + dl * _ETW])
            small_v[dl, pl.ds(_INV0 + c2 * 16, 16)] = vv


def _sc_body(
    pack_hbm, et_hbm, stage_hbm, small_hbm,
    in_a, in_b, et_v, sb_a, sb_b, sm_a, sm_b,
    si_a, si_b, ss_a, ss_b, sq_a, sq_b,
):
    cid = lax.axis_index("c")
    tid = lax.axis_index("s")
    dblk = tid % _NDB
    grp = tid // _NDB
    tb0 = (cid * 2 + grp) * _NTILE
    lane = lax.iota(jnp.int32, 16)

    pltpu.sync_copy(et_hbm.at[pl.ds(dblk * _DPT * _ETW, _DPT * _ETW)], et_v)

    bufs = ((in_a, sb_a, sm_a, si_a, ss_a, sq_a), (in_b, sb_b, sm_b, si_b, ss_b, sq_b))

    def in_src(tb):
        return pack_hbm.at[pl.ds(tb * _PACKW, _PACKW)]

    def stage_dst(tb):
        return stage_hbm.at[tb, pl.ds(dblk * _SROWS, _SROWS)]

    pltpu.async_copy(in_src(tb0), in_a, si_a)
    pltpu.async_copy(in_src(tb0 + 1), in_b, si_b)

    @pl.loop(0, _NTILE, step=2)
    def pair(i):
        for b in range(2):
            in_v, sbuf, small_v, si, ss, sq = bufs[b]
            g = i + b
            tb = tb0 + g
            pltpu.make_async_copy(in_src(tb), in_v, si).wait()

            @pl.when(g >= 2)
            def _wait_prev_out():
                pltpu.make_async_copy(sbuf, stage_dst(tb), ss).wait()
                pltpu.make_async_copy(small_v, small_hbm.at[tb, dblk], sq).wait()

            _compute_tb(lane, in_v, et_v, sbuf, small_v)

            pltpu.async_copy(sbuf, stage_dst(tb), ss)
            pltpu.async_copy(small_v, small_hbm.at[tb, dblk], sq)

            @pl.when(g + 2 < _NTILE)
            def _prefetch_next():
                pltpu.async_copy(in_src(tb + 2), in_v, si)

    for b in range(2):
        in_v, sbuf, small_v, si, ss, sq = bufs[b]
        tb = tb0 + _NTILE - 2 + b
        pltpu.make_async_copy(sbuf, stage_dst(tb), ss).wait()
        pltpu.make_async_copy(small_v, small_hbm.at[tb, dblk], sq).wait()


_TBS = 16


def _tc_depad_body(in_ref, out_ref):
    x = in_ref[:, :_R, :_C]
    out_ref[0] = x.reshape(_TBS, _D, _R, _C)


def _tc_depad(stage):
    planes = stage.reshape(_NTB * _D, _RP, _CT)
    return pl.pallas_call(
        _tc_depad_body,
        grid=(_NTB // _TBS,),
        in_specs=[
            pl.BlockSpec((_TBS * _D, _RP, _CT), lambda i: (i, 0, 0)),
        ],
        out_specs=pl.BlockSpec(
            (1, _TBS, _D, _R, _C),
            lambda i: ((i * _TBS) // _B, (i * _TBS) % _B // _TBS, 0, 0, 0),
        ),
        out_shape=jax.ShapeDtypeStruct((_T, _B, _D, _R, _C), jnp.float32),
    )(planes)


def kernel(glyphs, blstats, inv_glyphs, emb_table):
    gly = glyphs.reshape(_NTB, _R, _C).astype(jnp.int32)
    gly = jnp.pad(gly, ((0, 0), (0, 0), (0, _CP - _C)), constant_values=_MAXG)
    bl = blstats.reshape(_NTB, blstats.shape[-1]).astype(jnp.int32)
    inv = inv_glyphs.reshape(_NTB, _NINV).astype(jnp.int32)

    pack = jnp.full((_NTB, _PACKW), _MAXG, jnp.int32)
    pack = pack.at[:, :_GW].set(gly.reshape(_NTB, _GW))
    pack = pack.at[:, _META0].set(bl[:, 0])
    pack = pack.at[:, _META0 + 1].set(bl[:, 1])
    pack = pack.at[:, _INVO : _INVO + _NINV].set(inv)
    pack = pack.reshape(_NTB * _PACKW)

    et = jnp.zeros((_D, _ETW), jnp.float32)
    et = et.at[:, : _MAXG + 1].set(emb_table.astype(jnp.float32).T)
    et = et.reshape(_D * _ETW)

    mesh = plsc.VectorSubcoreMesh(
        core_axis_name="c", subcore_axis_name="s", num_cores=_NC, num_subcores=_NS
    )
    run = functools.partial(
        pl.kernel,
        out_type=[
            jax.ShapeDtypeStruct((_NTB, _NDB * _SROWS, _CT), jnp.float32),
            jax.ShapeDtypeStruct((_NTB, _NDB, _DPT, _CT), jnp.float32),
        ],
        mesh=mesh,
        compiler_params=pltpu.CompilerParams(
            needs_layout_passes=False, use_tc_tiling_on_sc=True
        ),
        scratch_types=[
            pltpu.VMEM((_PACKW,), jnp.int32),
            pltpu.VMEM((_PACKW,), jnp.int32),
            pltpu.VMEM((_DPT * _ETW,), jnp.float32),
            pltpu.VMEM((_SROWS, _CT), jnp.float32),
            pltpu.VMEM((_SROWS, _CT), jnp.float32),
            pltpu.VMEM((_DPT, _CT), jnp.float32),
            pltpu.VMEM((_DPT, _CT), jnp.float32),
            pltpu.SemaphoreType.DMA,
            pltpu.SemaphoreType.DMA,
            pltpu.SemaphoreType.DMA,
            pltpu.SemaphoreType.DMA,
            pltpu.SemaphoreType.DMA,
            pltpu.SemaphoreType.DMA,
        ],
    )(_sc_body)
    stage, small_f = run(pack, et)

    screen = _tc_depad(stage)
    vicinity = small_f[:, :, :, :_SELF0].reshape(_T, _B, _D, 3, 3)
    self_ = small_f[:, :, :, _SELF0].reshape(_T, _B, _D)
    inventory = small_f[:, :, :, _INV0 : _INV0 + _NINV].reshape(_T, _B, _D, _NINV)
    return screen, vicinity, inventory, self_
